# baseline (speedup 1.0000x reference)
import math

import jax
import jax.numpy as jnp
from jax.experimental import pallas as pl
from jax.experimental.pallas import tpu as pltpu

_CH = 2048
_NEG = -1e30


def _attn_kernel(starts_ref, ends_ref, enc_ref, wqt_ref, bq_ref, wkt_ref,
                 bk_ref, wvt_ref, bv_ref, soc_hbm, out_ref, bufs, sems):
    b, d = out_ref.shape
    t = soc_hbm.shape[0]
    nch = t // _CH

    def copy(i):
        return pltpu.make_async_copy(
            soc_hbm.at[pl.ds(i * _CH, _CH), :], bufs.at[i], sems.at[i])

    for i in range(nch):
        copy(i).start()

    q = jnp.dot(enc_ref[...], wqt_ref[...],
                preferred_element_type=jnp.float32) + bq_ref[...]
    q = jnp.maximum(q, 0.0) * (1.0 / math.sqrt(d))

    starts = starts_ref[...]
    ends = ends_ref[...]
    wkt, bk = wkt_ref[...], bk_ref[...]
    wvt, bv = wvt_ref[...], bv_ref[...]

    m = jnp.full((b, 1), _NEG, jnp.float32)
    s = jnp.zeros((b, 1), jnp.float32)
    acc = jnp.zeros((b, d), jnp.float32)

    for j in range(nch):
        copy(j).wait()
        tok = bufs[j]
        k = jnp.maximum(jnp.dot(tok, wkt,
                                preferred_element_type=jnp.float32) + bk, 0.0)
        v = jnp.maximum(jnp.dot(tok, wvt,
                                preferred_element_type=jnp.float32) + bv, 0.0)

        logits = jax.lax.dot_general(
            q, k, (((1,), (1,)), ((), ())),
            preferred_element_type=jnp.float32)
        col = j * _CH + jax.lax.broadcasted_iota(jnp.int32, (b, _CH), 1)
        mask = (col >= starts) & (col < ends)
        logits = jnp.where(mask, logits, _NEG)

        m_new = jnp.maximum(m, jnp.max(logits, axis=1, keepdims=True))
        alpha = jnp.exp(m - m_new)
        p = jnp.exp(logits - m_new)
        s = s * alpha + jnp.sum(p, axis=1, keepdims=True)
        acc = acc * alpha + jnp.dot(p, v, preferred_element_type=jnp.float32)
        m = m_new

    out_ref[...] = acc / s


def kernel(enc_hidden, social_ht, neighbors_idx_start, neighbors_idx_end,
           Wq, bq, Wk, bk, Wv, bv):
    b, d = enc_hidden.shape
    t = social_ht.shape[0]
    nch = t // _CH

    starts = neighbors_idx_start.astype(jnp.int32).reshape(b, 1)
    ends = neighbors_idx_end.astype(jnp.int32).reshape(b, 1)

    vmem = pl.BlockSpec(memory_space=pltpu.MemorySpace.VMEM)
    out = pl.pallas_call(
        _attn_kernel,
        in_specs=[vmem, vmem, vmem, vmem, vmem, vmem, vmem, vmem, vmem,
                  pl.BlockSpec(memory_space=pltpu.MemorySpace.HBM)],
        out_specs=vmem,
        out_shape=jax.ShapeDtypeStruct((b, d), jnp.float32),
        scratch_shapes=[
            pltpu.VMEM((nch, _CH, d), jnp.float32),
            pltpu.SemaphoreType.DMA((nch,)),
        ],
    )(starts, ends, enc_hidden,
      Wq.T, bq.reshape(1, d),
      Wk.T, bk.reshape(1, d),
      Wv.T, bv.reshape(1, d), social_ht)
    return out


# speedup vs baseline: 1.2209x; 1.2209x over previous
import math

import jax
import jax.numpy as jnp
from jax.experimental import pallas as pl
from jax.experimental.pallas import tpu as pltpu

_CH = 2048
_NEG = -1e30


def _attn_kernel(starts_ref, ends_ref, enc_ref, wq_ref, bq_ref, wk_ref,
                 bk_ref, wv_ref, bv_ref, soc_hbm, out_ref, bufs, sems):
    b, d = out_ref.shape
    t = soc_hbm.shape[0]
    nch = t // _CH

    def copy(i):
        return pltpu.make_async_copy(
            soc_hbm.at[pl.ds(i * _CH, _CH), :], bufs.at[i], sems.at[i])

    for i in range(nch):
        copy(i).start()

    _t = (((1,), (1,)), ((), ()))
    q = jax.lax.dot_general(enc_ref[...], wq_ref[...], _t,
                            preferred_element_type=jnp.float32) + bq_ref[...]
    q = jnp.maximum(q, 0.0) * (1.0 / math.sqrt(d))

    starts = starts_ref[...]
    ends = ends_ref[...]
    wk, bk = wk_ref[...], bk_ref[...]
    wv, bv = wv_ref[...], bv_ref[...]

    m = jnp.full((b, 1), _NEG, jnp.float32)
    s = jnp.zeros((b, 1), jnp.float32)
    acc = jnp.zeros((b, d), jnp.float32)

    for j in range(nch):
        copy(j).wait()
        tok = bufs[j]
        k = jnp.maximum(jax.lax.dot_general(
            tok, wk, _t, preferred_element_type=jnp.float32) + bk, 0.0)
        v = jnp.maximum(jax.lax.dot_general(
            tok, wv, _t, preferred_element_type=jnp.float32) + bv, 0.0)

        logits = jax.lax.dot_general(
            q, k, _t, preferred_element_type=jnp.float32)
        col = j * _CH + jax.lax.broadcasted_iota(jnp.int32, (b, _CH), 1)
        mask = (col >= starts) & (col < ends)
        logits = jnp.where(mask, logits, _NEG)

        m_new = jnp.maximum(m, jnp.max(logits, axis=1, keepdims=True))
        alpha = jnp.exp(m - m_new)
        p = jnp.exp(logits - m_new)
        s = s * alpha + jnp.sum(p, axis=1, keepdims=True)
        acc = acc * alpha + jnp.dot(p, v, preferred_element_type=jnp.float32)
        m = m_new

    out_ref[...] = acc / s


def kernel(enc_hidden, social_ht, neighbors_idx_start, neighbors_idx_end,
           Wq, bq, Wk, bk, Wv, bv):
    b, d = enc_hidden.shape
    t = social_ht.shape[0]
    nch = t // _CH

    starts = neighbors_idx_start.astype(jnp.int32).reshape(b, 1)
    ends = neighbors_idx_end.astype(jnp.int32).reshape(b, 1)

    vmem = pl.BlockSpec(memory_space=pltpu.MemorySpace.VMEM)
    out = pl.pallas_call(
        _attn_kernel,
        in_specs=[vmem, vmem, vmem, vmem, vmem, vmem, vmem, vmem, vmem,
                  pl.BlockSpec(memory_space=pltpu.MemorySpace.HBM)],
        out_specs=vmem,
        out_shape=jax.ShapeDtypeStruct((b, d), jnp.float32),
        scratch_shapes=[
            pltpu.VMEM((nch, _CH, d), jnp.float32),
            pltpu.SemaphoreType.DMA((nch,)),
        ],
    )(starts, ends, enc_hidden,
      Wq, bq.reshape(1, d),
      Wk, bk.reshape(1, d),
      Wv, bv.reshape(1, d), social_ht)
    return out
